# baseline (speedup 1.0000x reference)
import functools

import jax
import jax.numpy as jnp
from jax import lax
from jax.experimental import pallas as pl
from jax.experimental.pallas import tpu as pltpu
from jax.experimental.pallas import tpu_sc as plsc

N = 10000
E = 320000
D = 128
NC = 2
NS = 16
NW = NC * NS
CH = 128
NCHUNK = 80
EPW = NCHUNK * CH
NPHASE = 2
PCH = NCHUNK // NPHASE
NPAIR = PCH // 2
NPAD = 8
RPT = 624
TAIL = N - NS * RPT
TAIL0 = NS * RPT
BLK = 2000

_mesh = plsc.VectorSubcoreMesh(core_axis_name="c", subcore_axis_name="s")


@functools.partial(
    pl.kernel,
    out_type=jax.ShapeDtypeStruct((NC, N, D), jnp.float32),
    mesh=_mesh,
    scratch_types=[
        pltpu.VMEM_SHARED((N + NPAD, D), jnp.float32),
        pltpu.VMEM((PCH, CH), jnp.int32),
        pltpu.VMEM((CH, D), jnp.float32),
        pltpu.SemaphoreType.DMA,
    ],
)
def _deg_pass(dst3_h, ones_h, zeros_h, out_h, acc, dst_v, ones_v, sem):
    cid = lax.axis_index("c")
    sid = lax.axis_index("s")
    wid = cid * NS + sid
    r0 = sid * RPT
    pltpu.sync_copy(zeros_h.at[pl.ds(r0, RPT)], acc.at[pl.ds(r0, RPT)])

    @pl.when(sid == NS - 1)
    def _():
        pltpu.sync_copy(zeros_h.at[pl.ds(TAIL0, TAIL)],
                        acc.at[pl.ds(TAIL0, TAIL)])

    pltpu.sync_copy(ones_h, ones_v)
    plsc.subcore_barrier()

    for p in range(NPHASE):
        pltpu.sync_copy(dst3_h.at[wid, pl.ds(p * PCH, PCH)], dst_v)

        def body(j, c):
            pltpu.async_copy(ones_v, acc.at[dst_v.at[j]], sem, add=True)
            return c

        lax.fori_loop(0, PCH, body, 0, unroll=False)

        def drain(j, c):
            pltpu.make_async_copy(ones_v, acc.at[dst_v.at[0]], sem).wait()
            return c

        lax.fori_loop(0, PCH, drain, 0, unroll=False)

    plsc.subcore_barrier()
    pltpu.sync_copy(acc.at[pl.ds(r0, RPT)], out_h.at[cid, pl.ds(r0, RPT)])

    @pl.when(sid == NS - 1)
    def _():
        pltpu.sync_copy(acc.at[pl.ds(TAIL0, TAIL)],
                        out_h.at[cid, pl.ds(TAIL0, TAIL)])


@functools.partial(
    pl.kernel,
    out_type=jax.ShapeDtypeStruct((NC, N, D), jnp.float32),
    mesh=_mesh,
    scratch_types=[
        pltpu.VMEM_SHARED((N + NPAD, D), jnp.float32),
        pltpu.VMEM((PCH, CH), jnp.int32),
        pltpu.VMEM((PCH, CH), jnp.int32),
        pltpu.VMEM((CH, D), jnp.float32),
        pltpu.VMEM((CH, D), jnp.float32),
        pltpu.SemaphoreType.DMA,
        pltpu.SemaphoreType.DMA,
        pltpu.SemaphoreType.DMA,
        pltpu.SemaphoreType.DMA,
    ],
)
def _edge_pass(hs_h, src3_h, dst3_h, zeros_h, out_h, acc, src_v, dst_v,
               rows_a, rows_b, sem_ga, sem_gb, sem_sa, sem_sb):
    cid = lax.axis_index("c")
    sid = lax.axis_index("s")
    wid = cid * NS + sid
    r0 = sid * RPT
    pltpu.sync_copy(zeros_h.at[pl.ds(r0, RPT)], acc.at[pl.ds(r0, RPT)])

    @pl.when(sid == NS - 1)
    def _():
        pltpu.sync_copy(zeros_h.at[pl.ds(TAIL0, TAIL)],
                        acc.at[pl.ds(TAIL0, TAIL)])

    plsc.subcore_barrier()

    def gather(j, buf, sem):
        return pltpu.async_copy(hs_h.at[src_v.at[j]], buf, sem)

    def scatter(j, buf, sem):
        return pltpu.async_copy(buf, acc.at[dst_v.at[j]], sem, add=True)

    def gather_wait(j, buf, sem):
        pltpu.make_async_copy(hs_h.at[src_v.at[j]], buf, sem).wait()

    def scatter_wait(j, buf, sem):
        pltpu.make_async_copy(buf, acc.at[dst_v.at[j]], sem).wait()

    for p in range(NPHASE):
        pltpu.sync_copy(src3_h.at[wid, pl.ds(p * PCH, PCH)], src_v)
        pltpu.sync_copy(dst3_h.at[wid, pl.ds(p * PCH, PCH)], dst_v)
        gather(0, rows_a, sem_ga)

        def body(i, c):
            j0 = 2 * i
            j1 = j0 + 1
            gather_wait(j0, rows_a, sem_ga)

            @pl.when(i >= 1)
            def _():
                scatter_wait(j1 - 2, rows_b, sem_sb)

            gather(j1, rows_b, sem_gb)
            scatter(j0, rows_a, sem_sa)
            gather_wait(j1, rows_b, sem_gb)
            scatter_wait(j0, rows_a, sem_sa)

            @pl.when(i < NPAIR - 1)
            def _():
                gather(j0 + 2, rows_a, sem_ga)

            scatter(j1, rows_b, sem_sb)
            return c

        lax.fori_loop(0, NPAIR, body, 0, unroll=False)
        scatter_wait(PCH - 1, rows_b, sem_sb)

    plsc.subcore_barrier()
    pltpu.sync_copy(acc.at[pl.ds(r0, RPT)], out_h.at[cid, pl.ds(r0, RPT)])

    @pl.when(sid == NS - 1)
    def _():
        pltpu.sync_copy(acc.at[pl.ds(TAIL0, TAIL)],
                        out_h.at[cid, pl.ds(TAIL0, TAIL)])


def _dinv_of(deg_blk):
    return lax.rsqrt(1.0 + deg_blk[0, :, 0:1] + deg_blk[1, :, 0:1])


def _tc1_body(x_ref, wd_ref, bd_ref, w1_ref, deg_ref, ox_ref, h1s_ref):
    xb = x_ref[...]
    ox_ref[...] = (
        jnp.dot(xb, wd_ref[...], preferred_element_type=jnp.float32)
        + bd_ref[...]
    )
    dinv = _dinv_of(deg_ref[...])
    h1s_ref[...] = dinv * jnp.dot(
        xb, w1_ref[...], preferred_element_type=jnp.float32
    )


def _tc2_body(acc_ref, h1s_ref, deg_ref, b1_ref, w2_ref, h2s_ref):
    dinv = _dinv_of(deg_ref[...])
    a = acc_ref[...]
    s = a[0] + a[1] + h1s_ref[...]
    h1 = jnp.maximum(dinv * s + b1_ref[...], 0.0)
    h2s_ref[...] = dinv * jnp.dot(
        h1, w2_ref[...], preferred_element_type=jnp.float32
    )


def _tc3_body(acc_ref, h2s_ref, deg_ref, ox_ref, b2_ref, wdeg_ref, bdeg_ref,
              w3_ref, b3_ref, r1_ref, r2_ref, r3_ref):
    dinv = _dinv_of(deg_ref[...])
    a = acc_ref[...]
    h2 = dinv * (a[0] + a[1] + h2s_ref[...]) + b2_ref[...] + ox_ref[...]
    m = jnp.max(h2, axis=-1, keepdims=True)
    z = h2 - m
    r1_ref[...] = z - jnp.log(jnp.sum(jnp.exp(z), axis=-1, keepdims=True))
    r2_ref[...] = (
        jnp.dot(h2, wdeg_ref[...], preferred_element_type=jnp.float32)
        + bdeg_ref[...]
    )
    r3_ref[...] = (
        jnp.dot(h2, w3_ref[...], preferred_element_type=jnp.float32)
        + b3_ref[...]
    )


def _row_spec(w):
    return pl.BlockSpec((BLK, w), lambda i: (i, 0))


def _full_spec(r, c):
    return pl.BlockSpec((r, c), lambda i: (0, 0))


_deg_spec = pl.BlockSpec((NC, BLK, D), lambda i: (0, i, 0))
_acc_spec = pl.BlockSpec((NC, BLK, D), lambda i: (0, i, 0))
_grid = (N // BLK,)


def _tc1(x, wd, bd, w1, degp):
    return pl.pallas_call(
        _tc1_body,
        grid=_grid,
        in_specs=[_row_spec(D), _full_spec(D, D), _full_spec(1, D),
                  _full_spec(D, D), _deg_spec],
        out_specs=[_row_spec(D), _row_spec(D)],
        out_shape=[jax.ShapeDtypeStruct((N, D), jnp.float32)] * 2,
    )(x, wd, bd, w1, degp)


def _tc2(acc1, h1s, degp, b1, w2):
    return pl.pallas_call(
        _tc2_body,
        grid=_grid,
        in_specs=[_acc_spec, _row_spec(D), _deg_spec, _full_spec(1, D),
                  _full_spec(D, D)],
        out_specs=_row_spec(D),
        out_shape=jax.ShapeDtypeStruct((N, D), jnp.float32),
    )(acc1, h1s, degp, b1, w2)


def _tc3(acc2, h2s, degp, ox, b2, wdeg, bdeg, w3, b3):
    return pl.pallas_call(
        _tc3_body,
        grid=_grid,
        in_specs=[_acc_spec, _row_spec(D), _deg_spec, _row_spec(D),
                  _full_spec(1, D), _full_spec(D, 1), _full_spec(1, 1),
                  _full_spec(D, 32), _full_spec(1, 32)],
        out_specs=[_row_spec(D), _row_spec(1), _row_spec(32)],
        out_shape=[jax.ShapeDtypeStruct((N, D), jnp.float32),
                   jax.ShapeDtypeStruct((N, 1), jnp.float32),
                   jax.ShapeDtypeStruct((N, 32), jnp.float32)],
    )(acc2, h2s, degp, ox, b2, wdeg, bdeg, w3, b3)


def kernel(x, edge_index, W_down, b_down, W1, b1, W2, b2, W_deg, b_deg, W3,
           b3):
    npad = NW * NCHUNK * CH - E
    src3 = jnp.concatenate(
        [edge_index[0], jnp.zeros((npad,), jnp.int32)]).reshape(NW, NCHUNK, CH)
    dst3 = jnp.concatenate(
        [edge_index[1], jnp.full((npad,), N, jnp.int32)]).reshape(NW, NCHUNK,
                                                                  CH)
    zeros_row = jnp.zeros((N, D), jnp.float32)
    ones_deg = jnp.ones((CH, D), jnp.float32)

    degp = _deg_pass(dst3, ones_deg, zeros_row)
    ox, h1s = _tc1(x, W_down, b_down.reshape(1, D), W1, degp)
    src3z = jnp.zeros_like(src3)
    acc1 = _edge_pass(h1s, src3z, dst3, zeros_row)
    h2s = _tc2(acc1, h1s, degp, b1.reshape(1, D), W2)
    acc2 = _edge_pass(h2s, src3z, dst3, zeros_row)
    r1, r2, r3 = _tc3(acc2, h2s, degp, ox, b2.reshape(1, D),
                      W_deg, b_deg.reshape(1, 1), W3, b3.reshape(1, 32))
    return (r1, jnp.squeeze(r2, -1), r3)


# speedup vs baseline: 59.1838x; 59.1838x over previous
import functools

import jax
import jax.numpy as jnp
from jax import lax
from jax.experimental import pallas as pl
from jax.experimental.pallas import tpu as pltpu
from jax.experimental.pallas import tpu_sc as plsc

N = 10000
E = 320000
D = 128
NC = 2
NS = 16
NW = NC * NS
CH = 128
NCHUNK = 80
EPW = NCHUNK * CH
NPHASE = 2
PCH = NCHUNK // NPHASE
NPAIR = PCH // 2
NPAD = 8
RPT = 624
TAIL = N - NS * RPT
TAIL0 = NS * RPT
BLK = 2000

_mesh = plsc.VectorSubcoreMesh(core_axis_name="c", subcore_axis_name="s")


@functools.partial(
    pl.kernel,
    out_type=jax.ShapeDtypeStruct((NC, N, D), jnp.float32),
    mesh=_mesh,
    scratch_types=[
        pltpu.VMEM_SHARED((N + NPAD, D), jnp.float32),
        pltpu.VMEM((PCH, CH), jnp.int32),
        pltpu.VMEM((CH, D), jnp.float32),
        pltpu.SemaphoreType.DMA,
    ],
)
def _deg_pass(dst3_h, ones_h, zeros_h, out_h, acc, dst_v, ones_v, sem):
    cid = lax.axis_index("c")
    sid = lax.axis_index("s")
    wid = cid * NS + sid
    r0 = sid * RPT
    pltpu.sync_copy(zeros_h.at[pl.ds(r0, RPT)], acc.at[pl.ds(r0, RPT)])

    @pl.when(sid == NS - 1)
    def _():
        pltpu.sync_copy(zeros_h.at[pl.ds(TAIL0, TAIL)],
                        acc.at[pl.ds(TAIL0, TAIL)])

    pltpu.sync_copy(ones_h, ones_v)
    plsc.subcore_barrier()

    for p in range(NPHASE):
        pltpu.sync_copy(dst3_h.at[wid, pl.ds(p * PCH, PCH)], dst_v)

        def body(j, c):
            pltpu.async_copy(ones_v, acc.at[dst_v.at[j]], sem, add=True)
            return c

        lax.fori_loop(0, PCH, body, 0, unroll=False)

        def drain(j, c):
            pltpu.make_async_copy(ones_v, acc.at[dst_v.at[0]], sem).wait()
            return c

        lax.fori_loop(0, PCH, drain, 0, unroll=False)

    plsc.subcore_barrier()
    pltpu.sync_copy(acc.at[pl.ds(r0, RPT)], out_h.at[cid, pl.ds(r0, RPT)])

    @pl.when(sid == NS - 1)
    def _():
        pltpu.sync_copy(acc.at[pl.ds(TAIL0, TAIL)],
                        out_h.at[cid, pl.ds(TAIL0, TAIL)])


@functools.partial(
    pl.kernel,
    out_type=jax.ShapeDtypeStruct((NC, N, D), jnp.float32),
    mesh=_mesh,
    scratch_types=[
        pltpu.VMEM_SHARED((N + NPAD, D), jnp.float32),
        pltpu.VMEM((PCH, CH), jnp.int32),
        pltpu.VMEM((PCH, CH), jnp.int32),
        pltpu.VMEM((CH, D), jnp.float32),
        pltpu.VMEM((CH, D), jnp.float32),
        pltpu.SemaphoreType.DMA,
        pltpu.SemaphoreType.DMA,
        pltpu.SemaphoreType.DMA,
        pltpu.SemaphoreType.DMA,
    ],
)
def _edge_pass(hs_h, src3_h, dst3_h, zeros_h, out_h, acc, src_v, dst_v,
               rows_a, rows_b, sem_ga, sem_gb, sem_sa, sem_sb):
    cid = lax.axis_index("c")
    sid = lax.axis_index("s")
    wid = cid * NS + sid
    r0 = sid * RPT
    pltpu.sync_copy(zeros_h.at[pl.ds(r0, RPT)], acc.at[pl.ds(r0, RPT)])

    @pl.when(sid == NS - 1)
    def _():
        pltpu.sync_copy(zeros_h.at[pl.ds(TAIL0, TAIL)],
                        acc.at[pl.ds(TAIL0, TAIL)])

    plsc.subcore_barrier()

    def gather(j, buf, sem):
        return pltpu.async_copy(hs_h.at[src_v.at[j]], buf, sem)

    def scatter(j, buf, sem):
        return pltpu.async_copy(buf, acc.at[dst_v.at[j]], sem, add=True)

    def gather_wait(j, buf, sem):
        pltpu.make_async_copy(hs_h.at[src_v.at[j]], buf, sem).wait()

    def scatter_wait(j, buf, sem):
        pltpu.make_async_copy(buf, acc.at[dst_v.at[j]], sem).wait()

    for p in range(NPHASE):
        pltpu.sync_copy(src3_h.at[wid, pl.ds(p * PCH, PCH)], src_v)
        pltpu.sync_copy(dst3_h.at[wid, pl.ds(p * PCH, PCH)], dst_v)
        gather(0, rows_a, sem_ga)

        def body(i, c):
            j0 = 2 * i
            j1 = j0 + 1
            gather_wait(j0, rows_a, sem_ga)

            @pl.when(i >= 1)
            def _():
                scatter_wait(j1 - 2, rows_b, sem_sb)

            gather(j1, rows_b, sem_gb)
            scatter(j0, rows_a, sem_sa)
            gather_wait(j1, rows_b, sem_gb)
            scatter_wait(j0, rows_a, sem_sa)

            @pl.when(i < NPAIR - 1)
            def _():
                gather(j0 + 2, rows_a, sem_ga)

            scatter(j1, rows_b, sem_sb)
            return c

        lax.fori_loop(0, NPAIR, body, 0, unroll=False)
        scatter_wait(PCH - 1, rows_b, sem_sb)

    plsc.subcore_barrier()
    pltpu.sync_copy(acc.at[pl.ds(r0, RPT)], out_h.at[cid, pl.ds(r0, RPT)])

    @pl.when(sid == NS - 1)
    def _():
        pltpu.sync_copy(acc.at[pl.ds(TAIL0, TAIL)],
                        out_h.at[cid, pl.ds(TAIL0, TAIL)])


def _dinv_of(deg_blk):
    return lax.rsqrt(1.0 + deg_blk[0, :, 0:1] + deg_blk[1, :, 0:1])


def _tc1_body(x_ref, wd_ref, bd_ref, w1_ref, deg_ref, ox_ref, h1s_ref):
    xb = x_ref[...]
    ox_ref[...] = (
        jnp.dot(xb, wd_ref[...], preferred_element_type=jnp.float32)
        + bd_ref[...]
    )
    dinv = _dinv_of(deg_ref[...])
    h1s_ref[...] = dinv * jnp.dot(
        xb, w1_ref[...], preferred_element_type=jnp.float32
    )


def _tc2_body(acc_ref, h1s_ref, deg_ref, b1_ref, w2_ref, h2s_ref):
    dinv = _dinv_of(deg_ref[...])
    a = acc_ref[...]
    s = a[0] + a[1] + h1s_ref[...]
    h1 = jnp.maximum(dinv * s + b1_ref[...], 0.0)
    h2s_ref[...] = dinv * jnp.dot(
        h1, w2_ref[...], preferred_element_type=jnp.float32
    )


def _tc3_body(acc_ref, h2s_ref, deg_ref, ox_ref, b2_ref, wdeg_ref, bdeg_ref,
              w3_ref, b3_ref, r1_ref, r2_ref, r3_ref):
    dinv = _dinv_of(deg_ref[...])
    a = acc_ref[...]
    h2 = dinv * (a[0] + a[1] + h2s_ref[...]) + b2_ref[...] + ox_ref[...]
    m = jnp.max(h2, axis=-1, keepdims=True)
    z = h2 - m
    r1_ref[...] = z - jnp.log(jnp.sum(jnp.exp(z), axis=-1, keepdims=True))
    r2_ref[...] = (
        jnp.dot(h2, wdeg_ref[...], preferred_element_type=jnp.float32)
        + bdeg_ref[...]
    )
    r3_ref[...] = (
        jnp.dot(h2, w3_ref[...], preferred_element_type=jnp.float32)
        + b3_ref[...]
    )


def _row_spec(w):
    return pl.BlockSpec((BLK, w), lambda i: (i, 0))


def _full_spec(r, c):
    return pl.BlockSpec((r, c), lambda i: (0, 0))


_deg_spec = pl.BlockSpec((NC, BLK, D), lambda i: (0, i, 0))
_acc_spec = pl.BlockSpec((NC, BLK, D), lambda i: (0, i, 0))
_grid = (N // BLK,)


def _tc1(x, wd, bd, w1, degp):
    return pl.pallas_call(
        _tc1_body,
        grid=_grid,
        in_specs=[_row_spec(D), _full_spec(D, D), _full_spec(1, D),
                  _full_spec(D, D), _deg_spec],
        out_specs=[_row_spec(D), _row_spec(D)],
        out_shape=[jax.ShapeDtypeStruct((N, D), jnp.float32)] * 2,
    )(x, wd, bd, w1, degp)


def _tc2(acc1, h1s, degp, b1, w2):
    return pl.pallas_call(
        _tc2_body,
        grid=_grid,
        in_specs=[_acc_spec, _row_spec(D), _deg_spec, _full_spec(1, D),
                  _full_spec(D, D)],
        out_specs=_row_spec(D),
        out_shape=jax.ShapeDtypeStruct((N, D), jnp.float32),
    )(acc1, h1s, degp, b1, w2)


def _tc3(acc2, h2s, degp, ox, b2, wdeg, bdeg, w3, b3):
    return pl.pallas_call(
        _tc3_body,
        grid=_grid,
        in_specs=[_acc_spec, _row_spec(D), _deg_spec, _row_spec(D),
                  _full_spec(1, D), _full_spec(D, 1), _full_spec(1, 1),
                  _full_spec(D, 32), _full_spec(1, 32)],
        out_specs=[_row_spec(D), _row_spec(1), _row_spec(32)],
        out_shape=[jax.ShapeDtypeStruct((N, D), jnp.float32),
                   jax.ShapeDtypeStruct((N, 1), jnp.float32),
                   jax.ShapeDtypeStruct((N, 32), jnp.float32)],
    )(acc2, h2s, degp, ox, b2, wdeg, bdeg, w3, b3)


def kernel(x, edge_index, W_down, b_down, W1, b1, W2, b2, W_deg, b_deg, W3,
           b3):
    npad = NW * NCHUNK * CH - E
    src3 = jnp.concatenate(
        [edge_index[0],
         jnp.arange(npad, dtype=jnp.int32)]).reshape(NW, NCHUNK, CH)
    dst3 = jnp.concatenate(
        [edge_index[1], jnp.full((npad,), N, jnp.int32)]).reshape(NW, NCHUNK,
                                                                  CH)
    zeros_row = jnp.zeros((N, D), jnp.float32)
    ones_deg = jnp.ones((CH, D), jnp.float32)

    degp = _deg_pass(dst3, ones_deg, zeros_row)
    ox, h1s = _tc1(x, W_down, b_down.reshape(1, D), W1, degp)
    acc1 = _edge_pass(h1s, src3, dst3, zeros_row)
    h2s = _tc2(acc1, h1s, degp, b1.reshape(1, D), W2)
    acc2 = _edge_pass(h2s, src3, dst3, zeros_row)
    r1, r2, r3 = _tc3(acc2, h2s, degp, ox, b2.reshape(1, D),
                      W_deg, b_deg.reshape(1, 1), W3, b3.reshape(1, 32))
    return (r1, jnp.squeeze(r2, -1), r3)


# speedup vs baseline: 59.3455x; 1.0027x over previous
import functools

import jax
import jax.numpy as jnp
from jax import lax
from jax.experimental import pallas as pl
from jax.experimental.pallas import tpu as pltpu
from jax.experimental.pallas import tpu_sc as plsc

N = 10000
E = 320000
D = 128
NC = 2
NS = 16
NW = NC * NS
CH = 128
NCHUNK = 80
EPW = NCHUNK * CH
NPHASE = 2
PCH = NCHUNK // NPHASE
NPAIR = PCH // 2
NPAD = 8
RPT = 624
TAIL = N - NS * RPT
TAIL0 = NS * RPT
BLK = 2000

_mesh = plsc.VectorSubcoreMesh(core_axis_name="c", subcore_axis_name="s")


@functools.partial(
    pl.kernel,
    out_type=jax.ShapeDtypeStruct((NC, N, D), jnp.float32),
    mesh=_mesh,
    scratch_types=[
        pltpu.VMEM_SHARED((N + NPAD, D), jnp.float32),
        pltpu.VMEM((PCH, CH), jnp.int32),
        pltpu.VMEM((CH, D), jnp.float32),
        pltpu.SemaphoreType.DMA,
    ],
)
def _deg_pass(dst3_h, ones_h, zeros_h, out_h, acc, dst_v, ones_v, sem):
    cid = lax.axis_index("c")
    sid = lax.axis_index("s")
    wid = cid * NS + sid
    r0 = sid * RPT
    pltpu.sync_copy(zeros_h.at[pl.ds(r0, RPT)], acc.at[pl.ds(r0, RPT)])

    @pl.when(sid == NS - 1)
    def _():
        pltpu.sync_copy(zeros_h.at[pl.ds(TAIL0, TAIL)],
                        acc.at[pl.ds(TAIL0, TAIL)])

    pltpu.sync_copy(ones_h, ones_v)
    plsc.subcore_barrier()

    for p in range(NPHASE):
        pltpu.sync_copy(dst3_h.at[wid, pl.ds(p * PCH, PCH)], dst_v)

        def body(j, c):
            pltpu.async_copy(ones_v, acc.at[dst_v.at[j]], sem, add=True)
            return c

        lax.fori_loop(0, PCH, body, 0, unroll=False)

        def drain(j, c):
            pltpu.make_async_copy(ones_v, acc.at[dst_v.at[0]], sem).wait()
            return c

        lax.fori_loop(0, PCH, drain, 0, unroll=False)

    plsc.subcore_barrier()
    pltpu.sync_copy(acc.at[pl.ds(r0, RPT)], out_h.at[cid, pl.ds(r0, RPT)])

    @pl.when(sid == NS - 1)
    def _():
        pltpu.sync_copy(acc.at[pl.ds(TAIL0, TAIL)],
                        out_h.at[cid, pl.ds(TAIL0, TAIL)])


@functools.partial(
    pl.kernel,
    out_type=jax.ShapeDtypeStruct((NC, N, D), jnp.float32),
    mesh=_mesh,
    scratch_types=[
        pltpu.VMEM_SHARED((N + NPAD, D), jnp.float32),
        pltpu.VMEM((PCH, CH), jnp.int32),
        pltpu.VMEM((PCH, CH), jnp.int32),
        pltpu.VMEM((CH, D), jnp.float32),
        pltpu.VMEM((CH, D), jnp.float32),
        pltpu.SemaphoreType.DMA,
        pltpu.SemaphoreType.DMA,
        pltpu.SemaphoreType.DMA,
        pltpu.SemaphoreType.DMA,
    ],
)
def _edge_pass(hs_h, src3_h, dst3_h, zeros_h, out_h, acc, src_v, dst_v,
               rows_a, rows_b, sem_ga, sem_gb, sem_sa, sem_sb):
    cid = lax.axis_index("c")
    sid = lax.axis_index("s")
    wid = cid * NS + sid
    r0 = sid * RPT
    pltpu.sync_copy(zeros_h.at[pl.ds(r0, RPT)], acc.at[pl.ds(r0, RPT)])

    @pl.when(sid == NS - 1)
    def _():
        pltpu.sync_copy(zeros_h.at[pl.ds(TAIL0, TAIL)],
                        acc.at[pl.ds(TAIL0, TAIL)])

    plsc.subcore_barrier()

    def gather(j, buf, sem):
        return pltpu.async_copy(hs_h.at[src_v.at[j]], buf, sem)

    def scatter(j, buf, sem):
        return pltpu.async_copy(buf, acc.at[dst_v.at[j]], sem, add=True)

    def gather_wait(j, buf, sem):
        pltpu.make_async_copy(hs_h.at[src_v.at[j]], buf, sem).wait()

    def scatter_wait(j, buf, sem):
        pltpu.make_async_copy(buf, acc.at[dst_v.at[j]], sem).wait()

    for p in range(NPHASE):
        pltpu.sync_copy(src3_h.at[wid, pl.ds(p * PCH, PCH)], src_v)
        pltpu.sync_copy(dst3_h.at[wid, pl.ds(p * PCH, PCH)], dst_v)
        gather(0, rows_a, sem_ga)

        def body(i, c):
            j0 = 2 * i
            j1 = j0 + 1
            gather_wait(j0, rows_a, sem_ga)

            @pl.when(i >= 1)
            def _():
                scatter_wait(j1 - 2, rows_b, sem_sb)

            gather(j1, rows_b, sem_gb)
            scatter(j0, rows_a, sem_sa)
            gather_wait(j1, rows_b, sem_gb)
            scatter_wait(j0, rows_a, sem_sa)

            @pl.when(i < NPAIR - 1)
            def _():
                gather(j0 + 2, rows_a, sem_ga)

            scatter(j1, rows_b, sem_sb)
            return c

        lax.fori_loop(0, NPAIR, body, 0, unroll=False)
        scatter_wait(PCH - 1, rows_b, sem_sb)

    plsc.subcore_barrier()
    pltpu.sync_copy(acc.at[pl.ds(r0, RPT)], out_h.at[cid, pl.ds(r0, RPT)])

    @pl.when(sid == NS - 1)
    def _():
        pltpu.sync_copy(acc.at[pl.ds(TAIL0, TAIL)],
                        out_h.at[cid, pl.ds(TAIL0, TAIL)])


def _dinv_of(deg_blk):
    return lax.rsqrt(1.0 + deg_blk[0, :, 0:1] + deg_blk[1, :, 0:1])


def _tc1_body(x_ref, wd_ref, bd_ref, w1_ref, deg_ref, ox_ref, h1s_ref,
              dinv_ref):
    xb = x_ref[...]
    ox_ref[...] = (
        jnp.dot(xb, wd_ref[...], preferred_element_type=jnp.float32)
        + bd_ref[...]
    )
    dinv = _dinv_of(deg_ref[...])
    dinv_ref[...] = dinv
    h1s_ref[...] = dinv * jnp.dot(
        xb, w1_ref[...], preferred_element_type=jnp.float32
    )


def _tc2_body(acc_ref, h1s_ref, dinv_ref, b1_ref, w2_ref, h2s_ref):
    dinv = dinv_ref[...]
    a = acc_ref[...]
    s = a[0] + a[1] + h1s_ref[...]
    h1 = jnp.maximum(dinv * s + b1_ref[...], 0.0)
    h2s_ref[...] = dinv * jnp.dot(
        h1, w2_ref[...], preferred_element_type=jnp.float32
    )


def _tc3_body(acc_ref, h2s_ref, dinv_ref, ox_ref, b2_ref, wdeg_ref, bdeg_ref,
              w3_ref, b3_ref, r1_ref, r2_ref, r3_ref):
    dinv = dinv_ref[...]
    a = acc_ref[...]
    h2 = dinv * (a[0] + a[1] + h2s_ref[...]) + b2_ref[...] + ox_ref[...]
    m = jnp.max(h2, axis=-1, keepdims=True)
    z = h2 - m
    r1_ref[...] = z - jnp.log(jnp.sum(jnp.exp(z), axis=-1, keepdims=True))
    r2_ref[...] = (
        jnp.dot(h2, wdeg_ref[...], preferred_element_type=jnp.float32)
        + bdeg_ref[...]
    )
    r3_ref[...] = (
        jnp.dot(h2, w3_ref[...], preferred_element_type=jnp.float32)
        + b3_ref[...]
    )


def _row_spec(w):
    return pl.BlockSpec((BLK, w), lambda i: (i, 0))


def _full_spec(r, c):
    return pl.BlockSpec((r, c), lambda i: (0, 0))


_deg_spec = pl.BlockSpec((NC, BLK, D), lambda i: (0, i, 0))
_acc_spec = pl.BlockSpec((NC, BLK, D), lambda i: (0, i, 0))
_grid = (N // BLK,)


def _tc1(x, wd, bd, w1, degp):
    return pl.pallas_call(
        _tc1_body,
        grid=_grid,
        in_specs=[_row_spec(D), _full_spec(D, D), _full_spec(1, D),
                  _full_spec(D, D), _deg_spec],
        out_specs=[_row_spec(D), _row_spec(D), _row_spec(1)],
        out_shape=[jax.ShapeDtypeStruct((N, D), jnp.float32),
                   jax.ShapeDtypeStruct((N, D), jnp.float32),
                   jax.ShapeDtypeStruct((N, 1), jnp.float32)],
    )(x, wd, bd, w1, degp)


def _tc2(acc1, h1s, dinv, b1, w2):
    return pl.pallas_call(
        _tc2_body,
        grid=_grid,
        in_specs=[_acc_spec, _row_spec(D), _row_spec(1), _full_spec(1, D),
                  _full_spec(D, D)],
        out_specs=_row_spec(D),
        out_shape=jax.ShapeDtypeStruct((N, D), jnp.float32),
    )(acc1, h1s, dinv, b1, w2)


def _tc3(acc2, h2s, dinv, ox, b2, wdeg, bdeg, w3, b3):
    return pl.pallas_call(
        _tc3_body,
        grid=_grid,
        in_specs=[_acc_spec, _row_spec(D), _row_spec(1), _row_spec(D),
                  _full_spec(1, D), _full_spec(D, 1), _full_spec(1, 1),
                  _full_spec(D, 32), _full_spec(1, 32)],
        out_specs=[_row_spec(D), _row_spec(1), _row_spec(32)],
        out_shape=[jax.ShapeDtypeStruct((N, D), jnp.float32),
                   jax.ShapeDtypeStruct((N, 1), jnp.float32),
                   jax.ShapeDtypeStruct((N, 32), jnp.float32)],
    )(acc2, h2s, dinv, ox, b2, wdeg, bdeg, w3, b3)


def kernel(x, edge_index, W_down, b_down, W1, b1, W2, b2, W_deg, b_deg, W3,
           b3):
    npad = NW * NCHUNK * CH - E
    src3 = jnp.concatenate(
        [edge_index[0],
         jnp.arange(npad, dtype=jnp.int32)]).reshape(NW, NCHUNK, CH)
    dst3 = jnp.concatenate(
        [edge_index[1], jnp.full((npad,), N, jnp.int32)]).reshape(NW, NCHUNK,
                                                                  CH)
    zeros_row = jnp.zeros((N, D), jnp.float32)
    ones_deg = jnp.ones((CH, D), jnp.float32)

    degp = _deg_pass(dst3, ones_deg, zeros_row)
    ox, h1s, dinv = _tc1(x, W_down, b_down.reshape(1, D), W1, degp)
    acc1 = _edge_pass(h1s, src3, dst3, zeros_row)
    h2s = _tc2(acc1, h1s, dinv, b1.reshape(1, D), W2)
    acc2 = _edge_pass(h2s, src3, dst3, zeros_row)
    r1, r2, r3 = _tc3(acc2, h2s, dinv, ox, b2.reshape(1, D),
                      W_deg, b_deg.reshape(1, 1), W3, b3.reshape(1, 32))
    return (r1, jnp.squeeze(r2, -1), r3)
